# baseline (speedup 1.0000x reference)
import functools

import jax
import jax.numpy as jnp
from jax.experimental import pallas as pl
from jax.experimental.pallas import tpu as pltpu


def _seq_query_block(e_ref, q_ref, w1_ref, w2_ref, b1_ref, b2_ref, aw_ref,
                     ab_ref, out_ref, *, seg_per_block, seg_len):
    blk = pl.program_id(0)
    e = e_ref[...]
    q = q_ref[pl.ds(blk * seg_per_block, seg_per_block), :]
    qw = jax.lax.dot_general(q, w1_ref[...], (((1,), (1,)), ((), ())),
                             preferred_element_type=jnp.float32)
    qw = qw + b1_ref[...] + b2_ref[...]
    z = jax.lax.dot_general(e, w2_ref[...], (((1,), (1,)), ((), ())),
                            preferred_element_type=jnp.float32)
    rows = []
    for s in range(seg_per_block):
        lo = s * seg_len
        hs = jax.nn.sigmoid(z[lo:lo + seg_len] + qw[s:s + 1])
        es = e[lo:lo + seg_len]
        g = jax.lax.dot_general(hs, es, (((0,), (0,)), ((), ())),
                                preferred_element_type=jnp.float32)
        esum = jnp.sum(es, axis=0, keepdims=True)
        rows.append(
            jnp.dot(aw_ref[...], g, preferred_element_type=jnp.float32)
            + ab_ref[0, 0] * esum)
    out_ref[pl.ds(blk * seg_per_block, seg_per_block), :] = (
        jnp.concatenate(rows, axis=0))


def kernel(sess_embed, query, W1_w, W1_b, W2_w, W2_b, alpha_w, alpha_b,
           sections):
    N, d = sess_embed.shape
    B = query.shape[0]
    S = N // B
    SB = 2
    body = functools.partial(_seq_query_block, seg_per_block=SB, seg_len=S)

    return pl.pallas_call(
        body,
        grid=(B // SB,),
        in_specs=[
            pl.BlockSpec((SB * S, d), lambda b: (b, 0)),
            pl.BlockSpec((B, d), lambda b: (0, 0)),
            pl.BlockSpec((d, d), lambda b: (0, 0)),
            pl.BlockSpec((d, d), lambda b: (0, 0)),
            pl.BlockSpec((1, d), lambda b: (0, 0)),
            pl.BlockSpec((1, d), lambda b: (0, 0)),
            pl.BlockSpec((1, d), lambda b: (0, 0)),
            pl.BlockSpec((1, 1), lambda b: (0, 0)),
        ],
        out_specs=pl.BlockSpec((B, d), lambda b: (0, 0)),
        out_shape=jax.ShapeDtypeStruct((B, d), jnp.float32),
        compiler_params=pltpu.CompilerParams(
            dimension_semantics=("arbitrary",)),
    )(sess_embed, query, W1_w, W2_w, W1_b.reshape(1, d), W2_b.reshape(1, d),
      alpha_w, alpha_b.reshape(1, 1))


# speedup vs baseline: 1.0640x; 1.0640x over previous
import functools

import jax
import jax.numpy as jnp
from jax.experimental import pallas as pl
from jax.experimental.pallas import tpu as pltpu


def _seg_rows(e, qw, aw, ab, seg_off, nseg, seg_len, w2):
    z = jax.lax.dot_general(e, w2, (((1,), (1,)), ((), ())),
                            preferred_element_type=jnp.float32)
    rows = []
    for s in range(nseg):
        lo = s * seg_len
        hs = jax.nn.sigmoid(z[lo:lo + seg_len] + qw[seg_off + s:seg_off + s + 1])
        es = e[lo:lo + seg_len]
        g = jax.lax.dot_general(hs, es, (((0,), (0,)), ((), ())),
                                preferred_element_type=jnp.float32)
        esum = jnp.sum(es, axis=0, keepdims=True)
        rows.append(jnp.dot(aw, g, preferred_element_type=jnp.float32)
                    + ab * esum)
    return rows


def _seq_query_block(e1_ref, e2_ref, q_ref, w1_ref, w2_ref, b1_ref, b2_ref,
                     aw_ref, ab_ref, out_ref, *, seg_per_block, seg_len):
    blk = pl.program_id(0)
    half = seg_per_block // 2
    q = q_ref[pl.ds(blk * seg_per_block, seg_per_block), :]
    qw = jax.lax.dot_general(q, w1_ref[...], (((1,), (1,)), ((), ())),
                             preferred_element_type=jnp.float32)
    qw = qw + b1_ref[...] + b2_ref[...]
    aw = aw_ref[...]
    ab = ab_ref[0, 0]
    w2 = w2_ref[...]
    rows = _seg_rows(e1_ref[...], qw, aw, ab, 0, half, seg_len, w2)
    rows += _seg_rows(e2_ref[...], qw, aw, ab, half, seg_per_block - half,
                      seg_len, w2)
    out_ref[pl.ds(blk * seg_per_block, seg_per_block), :] = (
        jnp.concatenate(rows, axis=0))


def kernel(sess_embed, query, W1_w, W1_b, W2_w, W2_b, alpha_w, alpha_b,
           sections):
    N, d = sess_embed.shape
    B = query.shape[0]
    S = N // B
    SB = 4
    body = functools.partial(_seq_query_block, seg_per_block=SB, seg_len=S)
    half_rows = SB * S // 2

    return pl.pallas_call(
        body,
        grid=(B // SB,),
        in_specs=[
            pl.BlockSpec((half_rows, d), lambda b: (2 * b, 0)),
            pl.BlockSpec((half_rows, d), lambda b: (2 * b + 1, 0)),
            pl.BlockSpec((B, d), lambda b: (0, 0)),
            pl.BlockSpec((d, d), lambda b: (0, 0)),
            pl.BlockSpec((d, d), lambda b: (0, 0)),
            pl.BlockSpec((1, d), lambda b: (0, 0)),
            pl.BlockSpec((1, d), lambda b: (0, 0)),
            pl.BlockSpec((1, d), lambda b: (0, 0)),
            pl.BlockSpec((1, 1), lambda b: (0, 0)),
        ],
        out_specs=pl.BlockSpec((B, d), lambda b: (0, 0)),
        out_shape=jax.ShapeDtypeStruct((B, d), jnp.float32),
        compiler_params=pltpu.CompilerParams(
            dimension_semantics=("arbitrary",)),
    )(sess_embed, sess_embed, query, W1_w, W2_w, W1_b.reshape(1, d),
      W2_b.reshape(1, d), alpha_w, alpha_b.reshape(1, 1))


# speedup vs baseline: 1.0890x; 1.0235x over previous
import functools

import jax
import jax.numpy as jnp
from jax.experimental import pallas as pl
from jax.experimental.pallas import tpu as pltpu


def _seq_query_block(e_ref, q_ref, w1_ref, w2_ref, b1_ref, b2_ref, aw_ref,
                     ab_ref, out_ref, *, seg_per_block, seg_len):
    blk = pl.program_id(0)
    e = e_ref[...]
    q = q_ref[pl.ds(blk * seg_per_block, seg_per_block), :]
    qw = jax.lax.dot_general(q, w1_ref[...], (((1,), (1,)), ((), ())),
                             preferred_element_type=jnp.float32)
    qw = qw + b1_ref[...] + b2_ref[...]
    z = jax.lax.dot_general(e, w2_ref[...], (((1,), (1,)), ((), ())),
                            preferred_element_type=jnp.float32)
    rows = []
    for s in range(seg_per_block):
        lo = s * seg_len
        hs = 0.5 * jnp.tanh((z[lo:lo + seg_len] + qw[s:s + 1]) * 0.5) + 0.5
        es = e[lo:lo + seg_len]
        g = jax.lax.dot_general(hs, es, (((0,), (0,)), ((), ())),
                                preferred_element_type=jnp.float32)
        esum = jnp.sum(es, axis=0, keepdims=True)
        rows.append(
            jnp.dot(aw_ref[...], g, preferred_element_type=jnp.float32)
            + ab_ref[0, 0] * esum)
    out_ref[pl.ds(blk * seg_per_block, seg_per_block), :] = (
        jnp.concatenate(rows, axis=0))


def kernel(sess_embed, query, W1_w, W1_b, W2_w, W2_b, alpha_w, alpha_b,
           sections):
    N, d = sess_embed.shape
    B = query.shape[0]
    S = N // B
    SB = 4
    body = functools.partial(_seq_query_block, seg_per_block=SB, seg_len=S)

    return pl.pallas_call(
        body,
        grid=(B // SB,),
        in_specs=[
            pl.BlockSpec((SB * S, d), lambda b: (b, 0)),
            pl.BlockSpec((B, d), lambda b: (0, 0)),
            pl.BlockSpec((d, d), lambda b: (0, 0)),
            pl.BlockSpec((d, d), lambda b: (0, 0)),
            pl.BlockSpec((1, d), lambda b: (0, 0)),
            pl.BlockSpec((1, d), lambda b: (0, 0)),
            pl.BlockSpec((1, d), lambda b: (0, 0)),
            pl.BlockSpec((1, 1), lambda b: (0, 0)),
        ],
        out_specs=pl.BlockSpec((B, d), lambda b: (0, 0)),
        out_shape=jax.ShapeDtypeStruct((B, d), jnp.float32),
        compiler_params=pltpu.CompilerParams(
            dimension_semantics=("arbitrary",)),
    )(sess_embed, query, W1_w, W2_w, W1_b.reshape(1, d), W2_b.reshape(1, d),
      alpha_w, alpha_b.reshape(1, 1))
